# baseline (speedup 1.0000x reference)
import functools

import jax
import jax.numpy as jnp
from jax import lax
from jax.experimental import pallas as pl
from jax.experimental.pallas import tpu as pltpu
from jax.experimental.pallas import tpu_sc as plsc

B = 16384
D = 64
_INFO = plsc.get_sparse_core_info()
NC, NS, L = _INFO.num_cores, _INFO.num_subcores, _INFO.num_lanes
NW = NC * NS
BPW = B // NW
KCH = 4
CHUNK = BPW // KCH
GROUPS = BPW // L

_PIB = lax.GatherScatterMode.PROMISE_IN_BOUNDS
_DNUMS = lax.GatherDimensionNumbers(
    offset_dims=(), collapsed_slice_dims=(0,), start_index_map=(0,))


def _lane_gather(vec, idx):
    return lax.gather(vec, idx[:, None], _DNUMS, (1,), mode=_PIB)

_mesh = plsc.VectorSubcoreMesh(core_axis_name="c", subcore_axis_name="s")


@functools.partial(
    pl.kernel,
    mesh=_mesh,
    out_type=jax.ShapeDtypeStruct((B,), jnp.float32),
    compiler_params=pltpu.CompilerParams(
        use_tc_tiling_on_sc=False, needs_layout_passes=False),
    scratch_types=[
        pltpu.VMEM((KCH, CHUNK), jnp.int32),
        pltpu.VMEM((KCH, CHUNK), jnp.int32),
        pltpu.VMEM((BPW, D), jnp.float32),
        pltpu.VMEM((BPW, D), jnp.float32),
        pltpu.VMEM((2 * D,), jnp.float32),
        pltpu.VMEM((L,), jnp.float32),
        pltpu.VMEM((BPW,), jnp.float32),
        pltpu.SemaphoreType.DMA,
        pltpu.SemaphoreType.DMA,
    ],
)
def _sc_recsys(user_hbm, item_hbm, ut_hbm, it_hbm, w_hbm, b_hbm, out_hbm,
               uix, iix, rows_u, rows_i, w_v, b_v, out_v, sem_u, sem_i):
    wid = lax.axis_index("s") * NC + lax.axis_index("c")
    base = wid * BPW

    pltpu.sync_copy(user_hbm.at[wid], uix)
    pltpu.sync_copy(item_hbm.at[wid], iix)

    cu = [pltpu.async_copy(ut_hbm.at[uix.at[k]],
                           rows_u.at[pl.ds(k * CHUNK, CHUNK)], sem_u)
          for k in range(KCH)]
    ci = [pltpu.async_copy(it_hbm.at[iix.at[k]],
                           rows_i.at[pl.ds(k * CHUNK, CHUNK)], sem_i)
          for k in range(KCH)]

    pltpu.sync_copy(w_hbm, w_v)
    pltpu.sync_copy(b_hbm, b_v.at[pl.ds(0, 1)])

    iota = lax.iota(jnp.int32, L)
    zero16 = jnp.zeros((L,), jnp.int32)
    bias_bc = _lane_gather(b_v[...], zero16)

    def make_pass(rows_ref, w_off, init_from_out):
        def body(g, carry):
            rix = iota + g * L
            if init_from_out:
                acc = out_v[pl.ds(g * L, L)]
            else:
                acc = bias_bc
            for c in range(D // L):
                wv = w_v[pl.ds(w_off + c * L, L)]
                for jj in range(L):
                    j = c * L + jj
                    wbc = _lane_gather(wv, jnp.full((L,), jj, jnp.int32))
                    cix = jnp.full((L,), j, jnp.int32)
                    vals = plsc.load_gather(rows_ref, [rix, cix])
                    acc = acc + vals * wbc
            out_v[pl.ds(g * L, L)] = acc
            return carry
        return body

    for c in cu:
        c.wait()
    lax.fori_loop(0, GROUPS, make_pass(rows_u, 0, False), 0)
    for c in ci:
        c.wait()
    lax.fori_loop(0, GROUPS, make_pass(rows_i, D, True), 0)

    pltpu.sync_copy(out_v, out_hbm.at[pl.ds(base, BPW)])


def kernel(user, item, user_table, item_table, lin_w, lin_b):
    u3 = user.astype(jnp.int32).reshape(NW, KCH, CHUNK)
    i3 = item.astype(jnp.int32).reshape(NW, KCH, CHUNK)
    w = lin_w.reshape(2 * D)
    out = _sc_recsys(u3, i3, user_table, item_table, w, lin_b)
    return out.reshape(B, 1)


# speedup vs baseline: 1.5578x; 1.5578x over previous
import functools

import jax
import jax.numpy as jnp
from jax import lax
from jax.experimental import pallas as pl
from jax.experimental.pallas import tpu as pltpu
from jax.experimental.pallas import tpu_sc as plsc

B = 16384
D = 64
ROWW = 128
_INFO = plsc.get_sparse_core_info()
NC, NS, L = _INFO.num_cores, _INFO.num_subcores, _INFO.num_lanes
NW = NC * NS
BPW = B // NW
HALF = BPW // 2
HGRP = HALF // L

_mesh = plsc.VectorSubcoreMesh(core_axis_name="c", subcore_axis_name="s")

_DNUMS = lax.GatherDimensionNumbers(
    offset_dims=(), collapsed_slice_dims=(0,), start_index_map=(0,))
_PIB = lax.GatherScatterMode.PROMISE_IN_BOUNDS


def _lane_gather(vec, idx):
    return lax.gather(vec, idx[:, None], _DNUMS, (1,), mode=_PIB)


@functools.partial(
    pl.kernel,
    mesh=_mesh,
    out_type=jax.ShapeDtypeStruct((B,), jnp.float32),
    compiler_params=pltpu.CompilerParams(needs_layout_passes=False),
    scratch_types=[
        pltpu.VMEM((BPW,), jnp.int32),
        pltpu.VMEM((BPW,), jnp.int32),
        pltpu.VMEM((HALF, ROWW), jnp.float32),
        pltpu.VMEM((HALF, ROWW), jnp.float32),
        pltpu.VMEM((HALF, ROWW), jnp.float32),
        pltpu.VMEM((HALF * D,), jnp.float32),
        pltpu.VMEM((2 * D,), jnp.float32),
        pltpu.VMEM((L,), jnp.float32),
        pltpu.VMEM((BPW,), jnp.float32),
        pltpu.SemaphoreType.DMA,
        pltpu.SemaphoreType.DMA,
        pltpu.SemaphoreType.DMA,
    ],
)
def _sc_recsys(user_hbm, item_hbm, ut_hbm, it_hbm, w_hbm, b_hbm, out_hbm,
               uix, iix, buf_a, buf_b, buf_c, drain_v, w_v, b_v, out_v,
               sem_a, sem_b, sem_c):
    wid = lax.axis_index("s") * NC + lax.axis_index("c")
    base = wid * BPW
    iota = lax.iota(jnp.int32, L)

    pltpu.sync_copy(user_hbm.at[pl.ds(base, BPW)], uix)
    pltpu.sync_copy(item_hbm.at[pl.ds(base, BPW)], iix)

    def fire_block(idx_ref, table_ref, buf_ref, sem, blk):
        def body(g, carry):
            vec = idx_ref[pl.ds(blk * HALF + g * L, L)]
            for lane in range(L):
                i = jnp.max(jnp.where(iota == lane, vec, jnp.int32(0)))
                pltpu.async_copy(
                    table_ref.at[i], buf_ref.at[g * L + lane, pl.ds(0, D)],
                    sem)
            return carry
        lax.fori_loop(0, HGRP, body, 0)

    def drain(sem):
        pltpu.make_async_copy(out_hbm, drain_v, sem).wait()

    fire_block(uix, ut_hbm, buf_a, sem_a, 0)
    fire_block(uix, ut_hbm, buf_b, sem_b, 1)
    fire_block(iix, it_hbm, buf_c, sem_c, 0)

    pltpu.sync_copy(w_hbm.at[0], w_v)
    pltpu.sync_copy(b_hbm, b_v.at[pl.ds(0, 1)])
    bias_bc = _lane_gather(b_v[...], jnp.zeros((L,), jnp.int32))

    def compute_block(buf_ref, w_off, out_off, first):
        def body(g, carry):
            rix = iota + g * L
            if first:
                acc = bias_bc
            else:
                acc = out_v[pl.ds(out_off + g * L, L)]
            for c in range(D // L):
                wv = w_v[pl.ds(w_off + c * L, L)]
                for jj in range(L):
                    wbc = _lane_gather(wv, jnp.full((L,), jj, jnp.int32))
                    cix = jnp.full((L,), c * L + jj, jnp.int32)
                    vals = plsc.load_gather(buf_ref, [rix, cix])
                    acc = acc + vals * wbc
            out_v[pl.ds(out_off + g * L, L)] = acc
            return carry
        lax.fori_loop(0, HGRP, body, 0)

    drain(sem_a)
    compute_block(buf_a, 0, 0, True)
    fire_block(iix, it_hbm, buf_a, sem_a, 1)
    drain(sem_b)
    compute_block(buf_b, 0, HALF, True)
    drain(sem_c)
    compute_block(buf_c, D, 0, False)
    drain(sem_a)
    compute_block(buf_a, D, HALF, False)

    pltpu.sync_copy(out_v, out_hbm.at[pl.ds(base, BPW)])


def kernel(user, item, user_table, item_table, lin_w, lin_b):
    out = _sc_recsys(user.astype(jnp.int32), item.astype(jnp.int32),
                     user_table, item_table, lin_w, lin_b)
    return out.reshape(B, 1)
